# baseline (speedup 1.0000x reference)
import functools

import jax
import jax.numpy as jnp
from jax import lax
from jax.experimental import pallas as pl
from jax.experimental.pallas import tpu as pltpu
from jax.experimental.pallas import tpu_sc as plsc

N = 10000
NP = 10240
E = 320000
F = 128

NC = 2
NS = 16
NW = NC * NS
CH = 128
NCHUNK = NP // CH
EPW = NCHUNK * CH
EPAD = NW * EPW
RPT = NP // NS
RZ = RPT // CH

_MESH = plsc.VectorSubcoreMesh(
    core_axis_name="c", subcore_axis_name="s", num_cores=NC, num_subcores=NS
)


@functools.partial(
    pl.kernel,
    out_type=jax.ShapeDtypeStruct((2, NP, F), jnp.float32),
    mesh=_MESH,
    scratch_types=[
        pltpu.VMEM((2, NCHUNK, CH), jnp.int32),
        pltpu.VMEM((CH, F), jnp.float32),
        pltpu.VMEM_SHARED((NP, F), jnp.float32),
    ],
)
def _deg(src_hbm, dst_hbm, out_hbm, idx, buf, deg_sh):
    cid = lax.axis_index("c")
    sid = lax.axis_index("s")

    def zero_row(i, _):
        for j in range(F // 16):
            buf[i, pl.ds(j * 16, 16)] = jnp.zeros((16,), jnp.float32)
        return 0

    lax.fori_loop(0, CH, zero_row, 0)
    for r in range(RZ):
        pltpu.sync_copy(buf, deg_sh.at[pl.ds(sid * RPT + r * CH, CH)])
    plsc.subcore_barrier()

    def one_row(i, _):
        for j in range(F // 16):
            buf[i, pl.ds(j * 16, 16)] = jnp.ones((16,), jnp.float32)
        return 0

    lax.fori_loop(0, CH, one_row, 0)

    esl = pl.ds(2 * sid, 2)

    @pl.when(cid == 0)
    def _():
        pltpu.sync_copy(src_hbm.at[esl], idx)

    @pl.when(cid == 1)
    def _():
        pltpu.sync_copy(dst_hbm.at[esl], idx)

    for q in range(2):
        def body(k, _):
            pltpu.sync_copy(buf, deg_sh.at[idx.at[q, k]], add=True)
            return 0

        lax.fori_loop(0, NCHUNK, body, 0)
    plsc.subcore_barrier()

    sl = pl.ds(sid * RPT, RPT)
    pltpu.sync_copy(deg_sh.at[sl], out_hbm.at[cid, sl])


PAIR_CHUNKS = 2 * NCHUNK
SLAB = 40
NSLAB = PAIR_CHUNKS // SLAB
C0_SLABS = 1


@functools.partial(
    pl.kernel,
    out_type=jax.ShapeDtypeStruct((NC, NP, F), jnp.float32),
    mesh=_MESH,
    scratch_types=[
        pltpu.VMEM((SLAB, CH), jnp.int32),
        pltpu.VMEM((SLAB, CH), jnp.int32),
        pltpu.VMEM((CH, F), jnp.float32),
        pltpu.VMEM((CH, F), jnp.float32),
        pltpu.VMEM_SHARED((NP, F), jnp.float32),
        pltpu.SemaphoreType.DMA,
        pltpu.SemaphoreType.DMA,
    ],
)
def _scat(h_hbm, src_hbm, dst_hbm, out_hbm, sidx, didx, rows0, rows1, agg_sh,
          sem0, sem1):
    cid = lax.axis_index("c")
    sid = lax.axis_index("s")

    def zero_row(i, _):
        for j in range(F // 16):
            rows0[i, pl.ds(j * 16, 16)] = jnp.zeros((16,), jnp.float32)
        return 0

    lax.fori_loop(0, CH, zero_row, 0)
    for r in range(RZ):
        pltpu.sync_copy(rows0, agg_sh.at[pl.ds(sid * RPT + r * CH, CH)])
    plsc.subcore_barrier()

    NPAIR = SLAB // 2
    for s in range(NSLAB):
        owner = 0 if s < C0_SLABS else 1

        @pl.when(cid == owner)
        def _(s=s):
            pltpu.sync_copy(src_hbm.at[sid, pl.ds(s * SLAB, SLAB)], sidx)
            pltpu.sync_copy(dst_hbm.at[sid, pl.ds(s * SLAB, SLAB)], didx)
            pltpu.async_copy(h_hbm.at[sidx.at[0]], rows0, sem0)
            pltpu.async_copy(h_hbm.at[sidx.at[1]], rows1, sem1)

            def body(i, _):
                k0 = 2 * i
                pltpu.make_async_copy(h_hbm.at[sidx.at[k0]], rows0, sem0).wait()
                pltpu.sync_copy(rows0, agg_sh.at[didx.at[k0]], add=True)

                @pl.when(i + 1 < NPAIR)
                def _():
                    pltpu.async_copy(h_hbm.at[sidx.at[k0 + 2]], rows0, sem0)

                pltpu.make_async_copy(h_hbm.at[sidx.at[k0 + 1]], rows1, sem1).wait()
                pltpu.sync_copy(rows1, agg_sh.at[didx.at[k0 + 1]], add=True)

                @pl.when(i + 1 < NPAIR)
                def _():
                    pltpu.async_copy(h_hbm.at[sidx.at[k0 + 3]], rows1, sem1)

                return 0

            lax.fori_loop(0, NPAIR, body, 0)

    plsc.subcore_barrier()

    sl = pl.ds(sid * RPT, RPT)
    pltpu.sync_copy(agg_sh.at[sl], out_hbm.at[cid, sl])


_BLK = 1024
_GRID = NP // _BLK


def _tc_a_body(x_ref, w_ref, d_ref, h_ref, no_ref, ni_ref):
    d = d_ref[...]
    dego = d[0, :, 0:1]
    degi = d[1, :, 0:1]
    no = lax.rsqrt(jnp.clip(dego, 1.0, None))
    ni = lax.rsqrt(jnp.clip(degi, 1.0, None))
    y = jnp.dot(x_ref[...], w_ref[...], preferred_element_type=jnp.float32)
    h_ref[...] = y * no
    no_ref[...] = no
    ni_ref[...] = ni


def _tc_a(x, w1, degp):
    return pl.pallas_call(
        _tc_a_body,
        grid=(_GRID,),
        in_specs=[
            pl.BlockSpec((_BLK, F), lambda i: (i, 0)),
            pl.BlockSpec((F, F), lambda i: (0, 0)),
            pl.BlockSpec((2, _BLK, F), lambda i: (0, i, 0)),
        ],
        out_specs=[
            pl.BlockSpec((_BLK, F), lambda i: (i, 0)),
            pl.BlockSpec((_BLK, 1), lambda i: (i, 0)),
            pl.BlockSpec((_BLK, 1), lambda i: (i, 0)),
        ],
        out_shape=[
            jax.ShapeDtypeStruct((NP, F), jnp.float32),
            jax.ShapeDtypeStruct((NP, 1), jnp.float32),
            jax.ShapeDtypeStruct((NP, 1), jnp.float32),
        ],
    )(x, w1, degp)


def _tc_b_body(a_ref, ni_ref, no_ref, b_ref, w_ref, h_ref):
    a = a_ref[0] + a_ref[1]
    z = jnp.maximum(a * ni_ref[...] + b_ref[...], 0.0) * no_ref[...]
    h_ref[...] = jnp.dot(z, w_ref[...], preferred_element_type=jnp.float32)


def _tc_b(agg, ni, no, b1, w2):
    return pl.pallas_call(
        _tc_b_body,
        grid=(_GRID,),
        in_specs=[
            pl.BlockSpec((NC, _BLK, F), lambda i: (0, i, 0)),
            pl.BlockSpec((_BLK, 1), lambda i: (i, 0)),
            pl.BlockSpec((_BLK, 1), lambda i: (i, 0)),
            pl.BlockSpec((1, F), lambda i: (0, 0)),
            pl.BlockSpec((F, F), lambda i: (0, 0)),
        ],
        out_specs=pl.BlockSpec((_BLK, F), lambda i: (i, 0)),
        out_shape=jax.ShapeDtypeStruct((NP, F), jnp.float32),
    )(agg, ni, no, b1, w2)


def _tc_c_body(a_ref, ni_ref, b_ref, o_ref):
    a = a_ref[0] + a_ref[1]
    o_ref[...] = a * ni_ref[...] + b_ref[...]


def _tc_c(agg, ni, b2):
    return pl.pallas_call(
        _tc_c_body,
        grid=(_GRID,),
        in_specs=[
            pl.BlockSpec((NC, _BLK, F), lambda i: (0, i, 0)),
            pl.BlockSpec((_BLK, 1), lambda i: (i, 0)),
            pl.BlockSpec((1, F), lambda i: (0, 0)),
        ],
        out_specs=pl.BlockSpec((_BLK, F), lambda i: (i, 0)),
        out_shape=jax.ShapeDtypeStruct((NP, F), jnp.float32),
    )(agg, ni, b2)


def kernel(x, edge_index, W1, b1, W2, b2):
    pad = jnp.full((EPAD - E,), N, dtype=jnp.int32)
    src = jnp.concatenate([edge_index[0].astype(jnp.int32), pad])
    dst = jnp.concatenate([edge_index[1].astype(jnp.int32), pad])
    src = src.reshape(NW, NCHUNK, CH)
    dst = dst.reshape(NW, NCHUNK, CH)
    srcp = src.reshape(NS, PAIR_CHUNKS, CH)
    dstp = dst.reshape(NS, PAIR_CHUNKS, CH)
    xp = jnp.pad(x, ((0, NP - N), (0, 0)))

    degp = _deg(src, dst)
    h1s, no, ni = _tc_a(xp, W1, degp)
    agg1 = _scat(h1s, srcp, dstp)
    h2s = _tc_b(agg1, ni, no, b1.reshape(1, F), W2)
    agg2 = _scat(h2s, srcp, dstp)
    out = _tc_c(agg2, ni, b2.reshape(1, F))
    return out[:N]


# speedup vs baseline: 1.0979x; 1.0979x over previous
import functools

import jax
import jax.numpy as jnp
from jax import lax
from jax.experimental import pallas as pl
from jax.experimental.pallas import tpu as pltpu
from jax.experimental.pallas import tpu_sc as plsc

N = 10000
NP = 10240
E = 320000
F = 128

NC = 2
NS = 16
NW = NC * NS
CH = 128
NCHUNK = NP // CH
EPW = NCHUNK * CH
EPAD = NW * EPW
RPT = NP // NS
RZ = RPT // CH

_MESH = plsc.VectorSubcoreMesh(
    core_axis_name="c", subcore_axis_name="s", num_cores=NC, num_subcores=NS
)


@functools.partial(
    pl.kernel,
    out_type=jax.ShapeDtypeStruct((2, NP, F), jnp.float32),
    mesh=_MESH,
    scratch_types=[
        pltpu.VMEM((2, NCHUNK, CH), jnp.int32),
        pltpu.VMEM((CH, F), jnp.float32),
        pltpu.VMEM_SHARED((NP, F), jnp.float32),
    ],
)
def _deg(src_hbm, dst_hbm, out_hbm, idx, buf, deg_sh):
    cid = lax.axis_index("c")
    sid = lax.axis_index("s")

    def zero_row(i, _):
        for j in range(F // 16):
            buf[i, pl.ds(j * 16, 16)] = jnp.zeros((16,), jnp.float32)
        return 0

    lax.fori_loop(0, CH, zero_row, 0)
    for r in range(RZ):
        pltpu.sync_copy(buf, deg_sh.at[pl.ds(sid * RPT + r * CH, CH)])
    plsc.subcore_barrier()

    def one_row(i, _):
        for j in range(F // 16):
            buf[i, pl.ds(j * 16, 16)] = jnp.ones((16,), jnp.float32)
        return 0

    lax.fori_loop(0, CH, one_row, 0)

    esl = pl.ds(2 * sid, 2)

    @pl.when(cid == 0)
    def _():
        pltpu.sync_copy(src_hbm.at[esl], idx)

    @pl.when(cid == 1)
    def _():
        pltpu.sync_copy(dst_hbm.at[esl], idx)

    for q in range(2):
        def body(k, _):
            pltpu.sync_copy(buf, deg_sh.at[idx.at[q, k]], add=True)
            return 0

        lax.fori_loop(0, NCHUNK, body, 0)
    plsc.subcore_barrier()

    sl = pl.ds(sid * RPT, RPT)
    pltpu.sync_copy(deg_sh.at[sl], out_hbm.at[cid, sl])


PAIR_CHUNKS = 2 * NCHUNK
SLAB = 40
NSLAB = PAIR_CHUNKS // SLAB
C0_SLABS = 3


@functools.partial(
    pl.kernel,
    out_type=jax.ShapeDtypeStruct((NC, NP, F), jnp.float32),
    mesh=_MESH,
    scratch_types=[
        pltpu.VMEM((SLAB, CH), jnp.int32),
        pltpu.VMEM((SLAB, CH), jnp.int32),
        pltpu.VMEM((CH, F), jnp.float32),
        pltpu.VMEM((CH, F), jnp.float32),
        pltpu.VMEM_SHARED((NP, F), jnp.float32),
        pltpu.SemaphoreType.DMA,
        pltpu.SemaphoreType.DMA,
    ],
)
def _scat(h_hbm, src_hbm, dst_hbm, out_hbm, sidx, didx, rows0, rows1, agg_sh,
          sem0, sem1):
    cid = lax.axis_index("c")
    sid = lax.axis_index("s")

    def zero_row(i, _):
        for j in range(F // 16):
            rows0[i, pl.ds(j * 16, 16)] = jnp.zeros((16,), jnp.float32)
        return 0

    lax.fori_loop(0, CH, zero_row, 0)
    for r in range(RZ):
        pltpu.sync_copy(rows0, agg_sh.at[pl.ds(sid * RPT + r * CH, CH)])
    plsc.subcore_barrier()

    NPAIR = SLAB // 2
    for s in range(NSLAB):
        owner = 0 if s < C0_SLABS else 1

        @pl.when(cid == owner)
        def _(s=s):
            pltpu.sync_copy(src_hbm.at[sid, pl.ds(s * SLAB, SLAB)], sidx)
            pltpu.sync_copy(dst_hbm.at[sid, pl.ds(s * SLAB, SLAB)], didx)
            pltpu.async_copy(h_hbm.at[sidx.at[0]], rows0, sem0)
            pltpu.async_copy(h_hbm.at[sidx.at[1]], rows1, sem1)

            def body(i, _):
                k0 = 2 * i
                pltpu.make_async_copy(h_hbm.at[sidx.at[k0]], rows0, sem0).wait()
                pltpu.sync_copy(rows0, agg_sh.at[didx.at[k0]], add=True)

                @pl.when(i + 1 < NPAIR)
                def _():
                    pltpu.async_copy(h_hbm.at[sidx.at[k0 + 2]], rows0, sem0)

                pltpu.make_async_copy(h_hbm.at[sidx.at[k0 + 1]], rows1, sem1).wait()
                pltpu.sync_copy(rows1, agg_sh.at[didx.at[k0 + 1]], add=True)

                @pl.when(i + 1 < NPAIR)
                def _():
                    pltpu.async_copy(h_hbm.at[sidx.at[k0 + 3]], rows1, sem1)

                return 0

            lax.fori_loop(0, NPAIR, body, 0)

    plsc.subcore_barrier()

    sl = pl.ds(sid * RPT, RPT)
    pltpu.sync_copy(agg_sh.at[sl], out_hbm.at[cid, sl])


_BLK = 1024
_GRID = NP // _BLK


def _tc_a_body(x_ref, w_ref, d_ref, h_ref, no_ref, ni_ref):
    d = d_ref[...]
    dego = d[0, :, 0:1]
    degi = d[1, :, 0:1]
    no = lax.rsqrt(jnp.clip(dego, 1.0, None))
    ni = lax.rsqrt(jnp.clip(degi, 1.0, None))
    y = jnp.dot(x_ref[...], w_ref[...], preferred_element_type=jnp.float32)
    h_ref[...] = y * no
    no_ref[...] = no
    ni_ref[...] = ni


def _tc_a(x, w1, degp):
    return pl.pallas_call(
        _tc_a_body,
        grid=(_GRID,),
        in_specs=[
            pl.BlockSpec((_BLK, F), lambda i: (i, 0)),
            pl.BlockSpec((F, F), lambda i: (0, 0)),
            pl.BlockSpec((2, _BLK, F), lambda i: (0, i, 0)),
        ],
        out_specs=[
            pl.BlockSpec((_BLK, F), lambda i: (i, 0)),
            pl.BlockSpec((_BLK, 1), lambda i: (i, 0)),
            pl.BlockSpec((_BLK, 1), lambda i: (i, 0)),
        ],
        out_shape=[
            jax.ShapeDtypeStruct((NP, F), jnp.float32),
            jax.ShapeDtypeStruct((NP, 1), jnp.float32),
            jax.ShapeDtypeStruct((NP, 1), jnp.float32),
        ],
    )(x, w1, degp)


def _tc_b_body(a_ref, ni_ref, no_ref, b_ref, w_ref, h_ref):
    a = a_ref[0] + a_ref[1]
    z = jnp.maximum(a * ni_ref[...] + b_ref[...], 0.0) * no_ref[...]
    h_ref[...] = jnp.dot(z, w_ref[...], preferred_element_type=jnp.float32)


def _tc_b(agg, ni, no, b1, w2):
    return pl.pallas_call(
        _tc_b_body,
        grid=(_GRID,),
        in_specs=[
            pl.BlockSpec((NC, _BLK, F), lambda i: (0, i, 0)),
            pl.BlockSpec((_BLK, 1), lambda i: (i, 0)),
            pl.BlockSpec((_BLK, 1), lambda i: (i, 0)),
            pl.BlockSpec((1, F), lambda i: (0, 0)),
            pl.BlockSpec((F, F), lambda i: (0, 0)),
        ],
        out_specs=pl.BlockSpec((_BLK, F), lambda i: (i, 0)),
        out_shape=jax.ShapeDtypeStruct((NP, F), jnp.float32),
    )(agg, ni, no, b1, w2)


def _tc_c_body(a_ref, ni_ref, b_ref, o_ref):
    a = a_ref[0] + a_ref[1]
    o_ref[...] = a * ni_ref[...] + b_ref[...]


def _tc_c(agg, ni, b2):
    return pl.pallas_call(
        _tc_c_body,
        grid=(_GRID,),
        in_specs=[
            pl.BlockSpec((NC, _BLK, F), lambda i: (0, i, 0)),
            pl.BlockSpec((_BLK, 1), lambda i: (i, 0)),
            pl.BlockSpec((1, F), lambda i: (0, 0)),
        ],
        out_specs=pl.BlockSpec((_BLK, F), lambda i: (i, 0)),
        out_shape=jax.ShapeDtypeStruct((NP, F), jnp.float32),
    )(agg, ni, b2)


def kernel(x, edge_index, W1, b1, W2, b2):
    pad = jnp.full((EPAD - E,), N, dtype=jnp.int32)
    src = jnp.concatenate([edge_index[0].astype(jnp.int32), pad])
    dst = jnp.concatenate([edge_index[1].astype(jnp.int32), pad])
    src = src.reshape(NW, NCHUNK, CH)
    dst = dst.reshape(NW, NCHUNK, CH)
    srcp = src.reshape(NS, PAIR_CHUNKS, CH)
    dstp = dst.reshape(NS, PAIR_CHUNKS, CH)
    xp = jnp.pad(x, ((0, NP - N), (0, 0)))

    degp = _deg(src, dst)
    h1s, no, ni = _tc_a(xp, W1, degp)
    agg1 = _scat(h1s, srcp, dstp)
    h2s = _tc_b(agg1, ni, no, b1.reshape(1, F), W2)
    agg2 = _scat(h2s, srcp, dstp)
    out = _tc_c(agg2, ni, b2.reshape(1, F))
    return out[:N]


# speedup vs baseline: 2.4371x; 2.2199x over previous
import functools

import jax
import jax.numpy as jnp
from jax import lax
from jax.experimental import pallas as pl
from jax.experimental.pallas import tpu as pltpu
from jax.experimental.pallas import tpu_sc as plsc

N = 10000
NP = 10240
E = 320000
F = 128

NC = 2
NS = 16
NW = NC * NS
CH = 128
NCHUNK = NP // CH
EPW = NCHUNK * CH
EPAD = NW * EPW
RPT = NP // NS
RZ = RPT // CH

_MESH = plsc.VectorSubcoreMesh(
    core_axis_name="c", subcore_axis_name="s", num_cores=NC, num_subcores=NS
)


@functools.partial(
    pl.kernel,
    out_type=jax.ShapeDtypeStruct((2, NP, F), jnp.float32),
    mesh=_MESH,
    scratch_types=[
        pltpu.VMEM((2, NCHUNK, CH), jnp.int32),
        pltpu.VMEM((CH, F), jnp.float32),
        pltpu.VMEM_SHARED((NP, F), jnp.float32),
    ],
)
def _deg(src_hbm, dst_hbm, out_hbm, idx, buf, deg_sh):
    cid = lax.axis_index("c")
    sid = lax.axis_index("s")

    def zero_row(i, _):
        for j in range(F // 16):
            buf[i, pl.ds(j * 16, 16)] = jnp.zeros((16,), jnp.float32)
        return 0

    lax.fori_loop(0, CH, zero_row, 0)
    for r in range(RZ):
        pltpu.sync_copy(buf, deg_sh.at[pl.ds(sid * RPT + r * CH, CH)])
    plsc.subcore_barrier()

    def one_row(i, _):
        for j in range(F // 16):
            buf[i, pl.ds(j * 16, 16)] = jnp.ones((16,), jnp.float32)
        return 0

    lax.fori_loop(0, CH, one_row, 0)

    esl = pl.ds(2 * sid, 2)

    @pl.when(cid == 0)
    def _():
        pltpu.sync_copy(src_hbm.at[esl], idx)

    @pl.when(cid == 1)
    def _():
        pltpu.sync_copy(dst_hbm.at[esl], idx)

    for q in range(2):
        def body(k, _):
            pltpu.sync_copy(buf, deg_sh.at[idx.at[q, k]], add=True)
            return 0

        lax.fori_loop(0, NCHUNK, body, 0)
    plsc.subcore_barrier()

    sl = pl.ds(sid * RPT, RPT)
    pltpu.sync_copy(deg_sh.at[sl], out_hbm.at[cid, sl])


PAIR_CHUNKS = 2 * NCHUNK
SLAB = 40
NSLAB = PAIR_CHUNKS // SLAB
C0_SLABS = 2


@functools.partial(
    pl.kernel,
    out_type=jax.ShapeDtypeStruct((NC, NP, F), jnp.float32),
    mesh=_MESH,
    scratch_types=[
        pltpu.VMEM((SLAB, CH), jnp.int32),
        pltpu.VMEM((SLAB, CH), jnp.int32),
        pltpu.VMEM((CH, F), jnp.float32),
        pltpu.VMEM((CH, F), jnp.float32),
        pltpu.VMEM_SHARED((NP, F), jnp.float32),
        pltpu.SemaphoreType.DMA,
        pltpu.SemaphoreType.DMA,
    ],
)
def _scat(h_hbm, src_hbm, dst_hbm, out_hbm, sidx, didx, rows0, rows1, agg_sh,
          sem0, sem1):
    cid = lax.axis_index("c")
    sid = lax.axis_index("s")

    def zero_row(i, _):
        for j in range(F // 16):
            rows0[i, pl.ds(j * 16, 16)] = jnp.zeros((16,), jnp.float32)
        return 0

    lax.fori_loop(0, CH, zero_row, 0)
    for r in range(RZ):
        pltpu.sync_copy(rows0, agg_sh.at[pl.ds(sid * RPT + r * CH, CH)])
    plsc.subcore_barrier()

    NPAIR = SLAB // 2
    for s in range(NSLAB):
        owner = 0 if s < C0_SLABS else 1

        @pl.when(cid == owner)
        def _(s=s):
            pltpu.sync_copy(src_hbm.at[sid, pl.ds(s * SLAB, SLAB)], sidx)
            pltpu.sync_copy(dst_hbm.at[sid, pl.ds(s * SLAB, SLAB)], didx)
            pltpu.async_copy(h_hbm.at[sidx.at[0]], rows0, sem0)
            pltpu.async_copy(h_hbm.at[sidx.at[1]], rows1, sem1)

            def body(i, _):
                k0 = 2 * i
                pltpu.make_async_copy(h_hbm.at[sidx.at[k0]], rows0, sem0).wait()
                pltpu.sync_copy(rows0, agg_sh.at[didx.at[k0]], add=True)

                @pl.when(i + 1 < NPAIR)
                def _():
                    pltpu.async_copy(h_hbm.at[sidx.at[k0 + 2]], rows0, sem0)

                pltpu.make_async_copy(h_hbm.at[sidx.at[k0 + 1]], rows1, sem1).wait()
                pltpu.sync_copy(rows1, agg_sh.at[didx.at[k0 + 1]], add=True)

                @pl.when(i + 1 < NPAIR)
                def _():
                    pltpu.async_copy(h_hbm.at[sidx.at[k0 + 3]], rows1, sem1)

                return 0

            lax.fori_loop(0, NPAIR, body, 0)

    plsc.subcore_barrier()

    sl = pl.ds(sid * RPT, RPT)
    pltpu.sync_copy(agg_sh.at[sl], out_hbm.at[cid, sl])


_BLK = 1024
_GRID = NP // _BLK


def _tc_a_body(x_ref, w_ref, d_ref, h_ref, no_ref, ni_ref):
    d = d_ref[...]
    dego = d[0, :, 0:1]
    degi = d[1, :, 0:1]
    no = lax.rsqrt(jnp.clip(dego, 1.0, None))
    ni = lax.rsqrt(jnp.clip(degi, 1.0, None))
    y = jnp.dot(x_ref[...], w_ref[...], preferred_element_type=jnp.float32)
    h_ref[...] = y * no
    no_ref[...] = no
    ni_ref[...] = ni


def _tc_a(x, w1, degp):
    return pl.pallas_call(
        _tc_a_body,
        grid=(_GRID,),
        in_specs=[
            pl.BlockSpec((_BLK, F), lambda i: (i, 0)),
            pl.BlockSpec((F, F), lambda i: (0, 0)),
            pl.BlockSpec((2, _BLK, F), lambda i: (0, i, 0)),
        ],
        out_specs=[
            pl.BlockSpec((_BLK, F), lambda i: (i, 0)),
            pl.BlockSpec((_BLK, 1), lambda i: (i, 0)),
            pl.BlockSpec((_BLK, 1), lambda i: (i, 0)),
        ],
        out_shape=[
            jax.ShapeDtypeStruct((NP, F), jnp.float32),
            jax.ShapeDtypeStruct((NP, 1), jnp.float32),
            jax.ShapeDtypeStruct((NP, 1), jnp.float32),
        ],
    )(x, w1, degp)


def _tc_b_body(a_ref, ni_ref, no_ref, b_ref, w_ref, h_ref):
    a = a_ref[0] + a_ref[1]
    z = jnp.maximum(a * ni_ref[...] + b_ref[...], 0.0) * no_ref[...]
    h_ref[...] = jnp.dot(z, w_ref[...], preferred_element_type=jnp.float32)


def _tc_b(agg, ni, no, b1, w2):
    return pl.pallas_call(
        _tc_b_body,
        grid=(_GRID,),
        in_specs=[
            pl.BlockSpec((NC, _BLK, F), lambda i: (0, i, 0)),
            pl.BlockSpec((_BLK, 1), lambda i: (i, 0)),
            pl.BlockSpec((_BLK, 1), lambda i: (i, 0)),
            pl.BlockSpec((1, F), lambda i: (0, 0)),
            pl.BlockSpec((F, F), lambda i: (0, 0)),
        ],
        out_specs=pl.BlockSpec((_BLK, F), lambda i: (i, 0)),
        out_shape=jax.ShapeDtypeStruct((NP, F), jnp.float32),
    )(agg, ni, no, b1, w2)


def _tc_c_body(a_ref, ni_ref, b_ref, o_ref):
    a = a_ref[0] + a_ref[1]
    o_ref[...] = a * ni_ref[...] + b_ref[...]


def _tc_c(agg, ni, b2):
    return pl.pallas_call(
        _tc_c_body,
        grid=(_GRID,),
        in_specs=[
            pl.BlockSpec((NC, _BLK, F), lambda i: (0, i, 0)),
            pl.BlockSpec((_BLK, 1), lambda i: (i, 0)),
            pl.BlockSpec((1, F), lambda i: (0, 0)),
        ],
        out_specs=pl.BlockSpec((_BLK, F), lambda i: (i, 0)),
        out_shape=jax.ShapeDtypeStruct((NP, F), jnp.float32),
    )(agg, ni, b2)


def kernel(x, edge_index, W1, b1, W2, b2):
    pad = N + (jnp.arange(EPAD - E, dtype=jnp.int32) % (NP - N))
    src = jnp.concatenate([edge_index[0].astype(jnp.int32), pad])
    dst = jnp.concatenate([edge_index[1].astype(jnp.int32), pad])
    src = src.reshape(NW, NCHUNK, CH)
    dst = dst.reshape(NW, NCHUNK, CH)
    srcp = src.reshape(NS, PAIR_CHUNKS, CH)
    dstp = dst.reshape(NS, PAIR_CHUNKS, CH)
    xp = jnp.pad(x, ((0, NP - N), (0, 0)))

    degp = _deg(src, dst)
    h1s, no, ni = _tc_a(xp, W1, degp)
    agg1 = _scat(h1s, srcp, dstp)
    h2s = _tc_b(agg1, ni, no, b1.reshape(1, F), W2)
    agg2 = _scat(h2s, srcp, dstp)
    out = _tc_c(agg2, ni, b2.reshape(1, F))
    return out[:N]
